# baseline (speedup 1.0000x reference)
import functools

import jax
import jax.numpy as jnp
from jax import lax
from jax.experimental import pallas as pl
from jax.experimental.pallas import tpu as pltpu
from jax.experimental.pallas import tpu_sc as plsc

N = 10000
E = 320000
NC = 2
NS = 16
NW = NC * NS
CH = 128
NCH = 79
ECH = NW * NCH
EPAD = ECH * CH
NPAD = 10240
ROWS_PER_TILE = NPAD // NS
DW = 16

_SC_PARAMS = pltpu.CompilerParams(use_tc_tiling_on_sc=False)

AGG128_N0, AGG128_N1 = 94, 222
AGG64_N0, AGG64_N1 = 47, 111


@functools.lru_cache(maxsize=None)
def _mesh():
    return plsc.VectorSubcoreMesh(
        core_axis_name="c", subcore_axis_name="s",
        num_cores=NC, num_subcores=NS)


@functools.lru_cache(maxsize=None)
def _make_deg():
    return functools.partial(
        pl.kernel,
        out_type=jax.ShapeDtypeStruct((NC, NPAD, DW), jnp.float32),
        mesh=_mesh(),
        scratch_types=[
            pltpu.VMEM((NCH, CH), jnp.int32),
            pltpu.VMEM((NCH, CH), jnp.int32),
            pltpu.VMEM((CH, DW), jnp.float32),
            pltpu.VMEM((CH, DW), jnp.float32),
            pltpu.VMEM((ROWS_PER_TILE, DW), jnp.float32),
            pltpu.VMEM_SHARED((NPAD, DW), jnp.float32),
            pltpu.SemaphoreType.DMA,
            pltpu.SemaphoreType.DMA,
        ],
        compiler_params=_SC_PARAMS,
    )(_deg_body)


def _deg_body(src_hbm, dst_hbm, out_hbm, srcall, dstall, oneh0, oneh1, zbuf,
              dacc, semA, semB):
    cid = lax.axis_index("c")
    sid = lax.axis_index("s")
    wid = sid * NC + cid

    lane = lax.iota(jnp.int32, DW)
    e0 = jnp.where(lane == 0, 1.0, 0.0).astype(jnp.float32)
    e1 = jnp.where(lane == 1, 1.0, 0.0).astype(jnp.float32)
    z = jnp.zeros((DW,), jnp.float32)

    @pl.loop(0, CH)
    def _(r):
        oneh0[r, :] = e0
        oneh1[r, :] = e1

    @pl.loop(0, ROWS_PER_TILE)
    def _(r):
        zbuf[r, :] = z

    r0 = sid * ROWS_PER_TILE
    pltpu.sync_copy(zbuf, dacc.at[pl.ds(r0, ROWS_PER_TILE)])

    cbase = wid * NCH
    pltpu.sync_copy(src_hbm.at[pl.ds(cbase, NCH), :], srcall)
    pltpu.sync_copy(dst_hbm.at[pl.ds(cbase, NCH), :], dstall)
    plsc.subcore_barrier()

    def startA(c):
        pltpu.make_async_copy(oneh0, dacc.at[srcall.at[c]], semA).start(
            add=True)

    def startB(c):
        pltpu.make_async_copy(oneh1, dacc.at[dstall.at[c]], semB).start(
            add=True)

    def waitA(c):
        pltpu.make_async_copy(oneh0, dacc.at[srcall.at[c]], semA).wait()

    def waitB(c):
        pltpu.make_async_copy(oneh1, dacc.at[dstall.at[c]], semB).wait()

    startA(0)
    startB(0)

    @pl.loop(0, NCH - 1)
    def _(c):
        startA(c + 1)
        startB(c + 1)
        waitA(c)
        waitB(c)

    waitA(NCH - 1)
    waitB(NCH - 1)

    plsc.subcore_barrier()

    @pl.when(cid == 0)
    def _():
        pltpu.sync_copy(dacc.at[pl.ds(r0, ROWS_PER_TILE)],
                        out_hbm.at[0, pl.ds(r0, ROWS_PER_TILE), :])

    @pl.when(cid == 1)
    def _():
        pltpu.sync_copy(dacc.at[pl.ds(r0, ROWS_PER_TILE)],
                        out_hbm.at[1, pl.ds(r0, ROWS_PER_TILE), :])


@functools.lru_cache(maxsize=None)
def _make_agg(d, ch, n0, n1):
    assert NS * (n0 + n1) == EPAD // ch
    nmax = max(n0, n1)

    @functools.partial(
        pl.kernel,
        out_type=jax.ShapeDtypeStruct((NC, NPAD, d), jnp.float32),
        mesh=_mesh(),
        scratch_types=[
            pltpu.VMEM((nmax, ch), jnp.int32),
            pltpu.VMEM((ch,), jnp.int32),
            pltpu.VMEM((ch,), jnp.int32),
            pltpu.VMEM((ch, d), jnp.float32),
            pltpu.VMEM((ch, d), jnp.float32),
            pltpu.VMEM_SHARED((NPAD, d), jnp.float32),
            pltpu.SemaphoreType.DMA,
            pltpu.SemaphoreType.DMA,
            pltpu.SemaphoreType.DMA,
            pltpu.SemaphoreType.DMA,
        ],
        compiler_params=_SC_PARAMS,
    )
    def agg(y_hbm, src_hbm, dst_hbm, out_hbm, srcall, dstb0, dstb1, rows0,
            rows1, acc, gsem0, gsem1, dsem0, dsem1):
        cid = lax.axis_index("c")
        sid = lax.axis_index("s")

        z = jnp.zeros((16,), jnp.float32)

        @pl.loop(0, ch)
        def _(r):
            @pl.loop(0, d // 16)
            def _(j):
                rows0[r, pl.ds(j * 16, 16)] = z

        r0 = sid * ROWS_PER_TILE
        @pl.loop(0, ROWS_PER_TILE // ch)
        def _(k):
            pltpu.sync_copy(rows0, acc.at[pl.ds(r0 + k * ch, ch)])

        def run(cbase, count):
            pltpu.sync_copy(src_hbm.at[pl.ds(cbase, count), :],
                            srcall.at[pl.ds(0, count), :])

            def gstart(c, rows, sem):
                pltpu.make_async_copy(
                    y_hbm.at[srcall.at[c]], rows, sem).start()

            def gwait(c, rows, sem):
                pltpu.make_async_copy(y_hbm.at[srcall.at[c]], rows, sem).wait()

            def dstart(c, dstb, sem):
                pltpu.make_async_copy(
                    dst_hbm.at[cbase + c], dstb, sem).start()

            def dwait(c, dstb, sem):
                pltpu.make_async_copy(dst_hbm.at[cbase + c], dstb, sem).wait()

            def scat(rows, dstb):
                pltpu.sync_copy(rows, acc.at[dstb], add=True)

            dstart(0, dstb0, dsem0)
            gstart(0, rows0, gsem0)
            if count > 1:
                dstart(1, dstb1, dsem1)
                gstart(1, rows1, gsem1)

            @pl.loop(0, count // 2)
            def _(p):
                c0 = 2 * p
                gwait(c0, rows0, gsem0)
                dwait(c0, dstb0, dsem0)
                scat(rows0, dstb0)

                @pl.when(c0 + 2 < count)
                def _():
                    dstart(c0 + 2, dstb0, dsem0)
                    gstart(c0 + 2, rows0, gsem0)

                gwait(c0 + 1, rows1, gsem1)
                dwait(c0 + 1, dstb1, dsem1)
                scat(rows1, dstb1)

                @pl.when(c0 + 3 < count)
                def _():
                    dstart(c0 + 3, dstb1, dsem1)
                    gstart(c0 + 3, rows1, gsem1)

            if count % 2:
                gwait(count - 1, rows0, gsem0)
                dwait(count - 1, dstb0, dsem0)
                scat(rows0, dstb0)

        plsc.subcore_barrier()

        @pl.when(cid == 0)
        def _():
            run(sid * n0, n0)

        @pl.when(cid == 1)
        def _():
            run(NS * n0 + sid * n1, n1)

        plsc.subcore_barrier()

        @pl.when(cid == 0)
        def _():
            pltpu.sync_copy(acc.at[pl.ds(r0, ROWS_PER_TILE)],
                            out_hbm.at[0, pl.ds(r0, ROWS_PER_TILE), :])

        @pl.when(cid == 1)
        def _():
            pltpu.sync_copy(acc.at[pl.ds(r0, ROWS_PER_TILE)],
                            out_hbm.at[1, pl.ds(r0, ROWS_PER_TILE), :])

    return agg


_MB = 512
_GRIDP = NPAD // _MB


def _norm_body(dp_ref, nout_ref, nin_ref):
    s = dp_ref[0] + dp_ref[1]
    dout = s[:, 0:1]
    din = s[:, 1:2]
    nout_ref[...] = jnp.where(dout > 0, lax.rsqrt(dout), 0.0)
    nin_ref[...] = jnp.where(din > 0, lax.rsqrt(din), 0.0)


def _norms(dp):
    return pl.pallas_call(
        _norm_body,
        grid=(_GRIDP,),
        in_specs=[pl.BlockSpec((NC, _MB, DW), lambda i: (0, i, 0))],
        out_specs=[
            pl.BlockSpec((_MB, 1), lambda i: (i, 0)),
            pl.BlockSpec((_MB, 1), lambda i: (i, 0)),
        ],
        out_shape=[
            jax.ShapeDtypeStruct((NPAD, 1), jnp.float32),
            jax.ShapeDtypeStruct((NPAD, 1), jnp.float32),
        ],
    )(dp)


def _mm_first_body(x_ref, nout_ref, w_ref, y_ref):
    h = x_ref[...] * nout_ref[...]
    y_ref[...] = jnp.dot(h, w_ref[...], preferred_element_type=jnp.float32)


def _mm_first(x, nout, w):
    din, dout = w.shape
    return pl.pallas_call(
        _mm_first_body,
        grid=(_GRIDP,),
        in_specs=[
            pl.BlockSpec((_MB, din), lambda i: (i, 0)),
            pl.BlockSpec((_MB, 1), lambda i: (i, 0)),
            pl.BlockSpec((din, dout), lambda i: (0, 0)),
        ],
        out_specs=pl.BlockSpec((_MB, dout), lambda i: (i, 0)),
        out_shape=jax.ShapeDtypeStruct((NPAD, dout), jnp.float32),
    )(x, nout, w)


def _mm_mid_body(p_ref, nin_ref, b_ref, nout_ref, w_ref, y_ref):
    agg = p_ref[0] + p_ref[1]
    h = jax.nn.relu(agg * nin_ref[...] + b_ref[...])
    h = h * nout_ref[...]
    y_ref[...] = jnp.dot(h, w_ref[...], preferred_element_type=jnp.float32)


def _mm_mid(p, nin, b, nout, w):
    din, dout = w.shape
    return pl.pallas_call(
        _mm_mid_body,
        grid=(_GRIDP,),
        in_specs=[
            pl.BlockSpec((NC, _MB, din), lambda i: (0, i, 0)),
            pl.BlockSpec((_MB, 1), lambda i: (i, 0)),
            pl.BlockSpec((1, din), lambda i: (0, 0)),
            pl.BlockSpec((_MB, 1), lambda i: (i, 0)),
            pl.BlockSpec((din, dout), lambda i: (0, 0)),
        ],
        out_specs=pl.BlockSpec((_MB, dout), lambda i: (i, 0)),
        out_shape=jax.ShapeDtypeStruct((NPAD, dout), jnp.float32),
    )(p, nin, b, nout, w)


_FMB = 400


def _final_body(p_ref, nin_ref, b_ref, y_ref):
    agg = p_ref[0] + p_ref[1]
    y_ref[...] = agg * nin_ref[...] + b_ref[...]


def _final(p, nin, b):
    d = p.shape[-1]
    return pl.pallas_call(
        _final_body,
        grid=(N // _FMB,),
        in_specs=[
            pl.BlockSpec((NC, _FMB, d), lambda i: (0, i, 0)),
            pl.BlockSpec((_FMB, 1), lambda i: (i, 0)),
            pl.BlockSpec((1, d), lambda i: (0, 0)),
        ],
        out_specs=pl.BlockSpec((_FMB, d), lambda i: (i, 0)),
        out_shape=jax.ShapeDtypeStruct((N, d), jnp.float32),
    )(p, nin, b)


def kernel(inputs, edge_index, W0, b0, W1, b1, W2, b2):
    ei = edge_index.astype(jnp.int32)
    pad = jnp.full((2, EPAD - E), N, jnp.int32)
    ei = jnp.concatenate([ei, pad], axis=1)
    src = ei[0].reshape(ECH, CH)
    dst = ei[1].reshape(ECH, CH)
    src64 = ei[0].reshape(ECH * 2, CH // 2)
    dst64 = ei[1].reshape(ECH * 2, CH // 2)

    xp = jnp.pad(inputs, ((0, NPAD - N), (0, 0)))

    dp = _make_deg()(src, dst)
    nout, nin = _norms(dp)

    agg128 = _make_agg(128, 64, AGG128_N0, AGG128_N1)
    y0 = _mm_first(xp, nout, W0)
    p0 = agg128(y0, src64, dst64)
    y1 = _mm_mid(p0, nin, b0.reshape(1, -1), nout, W1)
    p1 = agg128(y1, src64, dst64)
    y2 = _mm_mid(p1, nin, b1.reshape(1, -1), nout, W2)
    p2 = _make_agg(64, 128, AGG64_N0, AGG64_N1)(y2, src, dst)
    return _final(p2, nin, b2.reshape(1, -1))


# speedup vs baseline: 1.2362x; 1.2362x over previous
import functools

import jax
import jax.numpy as jnp
from jax import lax
from jax.experimental import pallas as pl
from jax.experimental.pallas import tpu as pltpu
from jax.experimental.pallas import tpu_sc as plsc

N = 10000
E = 320000
NC = 2
NS = 16
NW = NC * NS
CH = 128
NCH = 79
ECH = NW * NCH
EPAD = ECH * CH
NPAD = 10240
ROWS_PER_TILE = NPAD // NS
DW = 16

_SC_PARAMS = pltpu.CompilerParams(use_tc_tiling_on_sc=False)

AGG128_N0, AGG128_N1 = 222, 94
AGG64_N0, AGG64_N1 = 111, 47


@functools.lru_cache(maxsize=None)
def _mesh():
    return plsc.VectorSubcoreMesh(
        core_axis_name="c", subcore_axis_name="s",
        num_cores=NC, num_subcores=NS)


@functools.lru_cache(maxsize=None)
def _make_deg():
    return functools.partial(
        pl.kernel,
        out_type=jax.ShapeDtypeStruct((NC, NPAD, DW), jnp.float32),
        mesh=_mesh(),
        scratch_types=[
            pltpu.VMEM((NCH, CH), jnp.int32),
            pltpu.VMEM((NCH, CH), jnp.int32),
            pltpu.VMEM((CH, DW), jnp.float32),
            pltpu.VMEM((CH, DW), jnp.float32),
            pltpu.VMEM((ROWS_PER_TILE, DW), jnp.float32),
            pltpu.VMEM_SHARED((NPAD, DW), jnp.float32),
            pltpu.SemaphoreType.DMA,
            pltpu.SemaphoreType.DMA,
        ],
        compiler_params=_SC_PARAMS,
    )(_deg_body)


def _deg_body(src_hbm, dst_hbm, out_hbm, srcall, dstall, oneh0, oneh1, zbuf,
              dacc, semA, semB):
    cid = lax.axis_index("c")
    sid = lax.axis_index("s")
    wid = sid * NC + cid

    lane = lax.iota(jnp.int32, DW)
    e0 = jnp.where(lane == 0, 1.0, 0.0).astype(jnp.float32)
    e1 = jnp.where(lane == 1, 1.0, 0.0).astype(jnp.float32)
    z = jnp.zeros((DW,), jnp.float32)

    @pl.loop(0, CH)
    def _(r):
        oneh0[r, :] = e0
        oneh1[r, :] = e1

    @pl.loop(0, ROWS_PER_TILE)
    def _(r):
        zbuf[r, :] = z

    r0 = sid * ROWS_PER_TILE
    pltpu.sync_copy(zbuf, dacc.at[pl.ds(r0, ROWS_PER_TILE)])

    cbase = wid * NCH
    pltpu.sync_copy(src_hbm.at[pl.ds(cbase, NCH), :], srcall)
    pltpu.sync_copy(dst_hbm.at[pl.ds(cbase, NCH), :], dstall)
    plsc.subcore_barrier()

    def startA(c):
        pltpu.make_async_copy(oneh0, dacc.at[srcall.at[c]], semA).start(
            add=True)

    def startB(c):
        pltpu.make_async_copy(oneh1, dacc.at[dstall.at[c]], semB).start(
            add=True)

    def waitA(c):
        pltpu.make_async_copy(oneh0, dacc.at[srcall.at[c]], semA).wait()

    def waitB(c):
        pltpu.make_async_copy(oneh1, dacc.at[dstall.at[c]], semB).wait()

    startA(0)
    startB(0)

    @pl.loop(0, NCH - 1)
    def _(c):
        startA(c + 1)
        startB(c + 1)
        waitA(c)
        waitB(c)

    waitA(NCH - 1)
    waitB(NCH - 1)

    plsc.subcore_barrier()

    @pl.when(cid == 0)
    def _():
        pltpu.sync_copy(dacc.at[pl.ds(r0, ROWS_PER_TILE)],
                        out_hbm.at[0, pl.ds(r0, ROWS_PER_TILE), :])

    @pl.when(cid == 1)
    def _():
        pltpu.sync_copy(dacc.at[pl.ds(r0, ROWS_PER_TILE)],
                        out_hbm.at[1, pl.ds(r0, ROWS_PER_TILE), :])


@functools.lru_cache(maxsize=None)
def _make_agg(d, ch, n0, n1):
    assert NS * (n0 + n1) == EPAD // ch
    nmax = max(n0, n1)

    @functools.partial(
        pl.kernel,
        out_type=jax.ShapeDtypeStruct((NC, NPAD, d), jnp.float32),
        mesh=_mesh(),
        scratch_types=[
            pltpu.VMEM((nmax, ch), jnp.int32),
            pltpu.VMEM((ch,), jnp.int32),
            pltpu.VMEM((ch,), jnp.int32),
            pltpu.VMEM((ch, d), jnp.float32),
            pltpu.VMEM((ch, d), jnp.float32),
            pltpu.VMEM_SHARED((NPAD, d), jnp.float32),
            pltpu.SemaphoreType.DMA,
            pltpu.SemaphoreType.DMA,
            pltpu.SemaphoreType.DMA,
            pltpu.SemaphoreType.DMA,
        ],
        compiler_params=_SC_PARAMS,
    )
    def agg(y_hbm, src_hbm, dst_hbm, out_hbm, srcall, dstb0, dstb1, rows0,
            rows1, acc, gsem0, gsem1, dsem0, dsem1):
        cid = lax.axis_index("c")
        sid = lax.axis_index("s")

        z = jnp.zeros((16,), jnp.float32)

        @pl.loop(0, ch)
        def _(r):
            @pl.loop(0, d // 16)
            def _(j):
                rows0[r, pl.ds(j * 16, 16)] = z

        r0 = sid * ROWS_PER_TILE
        @pl.loop(0, ROWS_PER_TILE // ch)
        def _(k):
            pltpu.sync_copy(rows0, acc.at[pl.ds(r0 + k * ch, ch)])

        def run(cbase, count):
            pltpu.sync_copy(src_hbm.at[pl.ds(cbase, count), :],
                            srcall.at[pl.ds(0, count), :])

            def gstart(c, rows, sem):
                pltpu.make_async_copy(
                    y_hbm.at[srcall.at[c]], rows, sem).start()

            def gwait(c, rows, sem):
                pltpu.make_async_copy(y_hbm.at[srcall.at[c]], rows, sem).wait()

            def dstart(c, dstb, sem):
                pltpu.make_async_copy(
                    dst_hbm.at[cbase + c], dstb, sem).start()

            def dwait(c, dstb, sem):
                pltpu.make_async_copy(dst_hbm.at[cbase + c], dstb, sem).wait()

            def scat(rows, dstb):
                pltpu.sync_copy(rows, acc.at[dstb], add=True)

            dstart(0, dstb0, dsem0)
            gstart(0, rows0, gsem0)
            if count > 1:
                dstart(1, dstb1, dsem1)
                gstart(1, rows1, gsem1)

            @pl.loop(0, count // 2)
            def _(p):
                c0 = 2 * p
                gwait(c0, rows0, gsem0)
                dwait(c0, dstb0, dsem0)
                scat(rows0, dstb0)

                @pl.when(c0 + 2 < count)
                def _():
                    dstart(c0 + 2, dstb0, dsem0)
                    gstart(c0 + 2, rows0, gsem0)

                gwait(c0 + 1, rows1, gsem1)
                dwait(c0 + 1, dstb1, dsem1)
                scat(rows1, dstb1)

                @pl.when(c0 + 3 < count)
                def _():
                    dstart(c0 + 3, dstb1, dsem1)
                    gstart(c0 + 3, rows1, gsem1)

            if count % 2:
                gwait(count - 1, rows0, gsem0)
                dwait(count - 1, dstb0, dsem0)
                scat(rows0, dstb0)

        plsc.subcore_barrier()

        @pl.when(cid == 0)
        def _():
            run(sid * n0, n0)

        @pl.when(cid == 1)
        def _():
            run(NS * n0 + sid * n1, n1)

        plsc.subcore_barrier()

        @pl.when(cid == 0)
        def _():
            pltpu.sync_copy(acc.at[pl.ds(r0, ROWS_PER_TILE)],
                            out_hbm.at[0, pl.ds(r0, ROWS_PER_TILE), :])

        @pl.when(cid == 1)
        def _():
            pltpu.sync_copy(acc.at[pl.ds(r0, ROWS_PER_TILE)],
                            out_hbm.at[1, pl.ds(r0, ROWS_PER_TILE), :])

    return agg


_MB = 512
_GRIDP = NPAD // _MB


def _norm_body(dp_ref, nout_ref, nin_ref):
    s = dp_ref[0] + dp_ref[1]
    dout = s[:, 0:1]
    din = s[:, 1:2]
    nout_ref[...] = jnp.where(dout > 0, lax.rsqrt(dout), 0.0)
    nin_ref[...] = jnp.where(din > 0, lax.rsqrt(din), 0.0)


def _norms(dp):
    return pl.pallas_call(
        _norm_body,
        grid=(_GRIDP,),
        in_specs=[pl.BlockSpec((NC, _MB, DW), lambda i: (0, i, 0))],
        out_specs=[
            pl.BlockSpec((_MB, 1), lambda i: (i, 0)),
            pl.BlockSpec((_MB, 1), lambda i: (i, 0)),
        ],
        out_shape=[
            jax.ShapeDtypeStruct((NPAD, 1), jnp.float32),
            jax.ShapeDtypeStruct((NPAD, 1), jnp.float32),
        ],
    )(dp)


def _mm_first_body(x_ref, nout_ref, w_ref, y_ref):
    h = x_ref[...] * nout_ref[...]
    y_ref[...] = jnp.dot(h, w_ref[...], preferred_element_type=jnp.float32)


def _mm_first(x, nout, w):
    din, dout = w.shape
    return pl.pallas_call(
        _mm_first_body,
        grid=(_GRIDP,),
        in_specs=[
            pl.BlockSpec((_MB, din), lambda i: (i, 0)),
            pl.BlockSpec((_MB, 1), lambda i: (i, 0)),
            pl.BlockSpec((din, dout), lambda i: (0, 0)),
        ],
        out_specs=pl.BlockSpec((_MB, dout), lambda i: (i, 0)),
        out_shape=jax.ShapeDtypeStruct((NPAD, dout), jnp.float32),
    )(x, nout, w)


def _mm_mid_body(p_ref, nin_ref, b_ref, nout_ref, w_ref, y_ref):
    agg = p_ref[0] + p_ref[1]
    h = jax.nn.relu(agg * nin_ref[...] + b_ref[...])
    h = h * nout_ref[...]
    y_ref[...] = jnp.dot(h, w_ref[...], preferred_element_type=jnp.float32)


def _mm_mid(p, nin, b, nout, w):
    din, dout = w.shape
    return pl.pallas_call(
        _mm_mid_body,
        grid=(_GRIDP,),
        in_specs=[
            pl.BlockSpec((NC, _MB, din), lambda i: (0, i, 0)),
            pl.BlockSpec((_MB, 1), lambda i: (i, 0)),
            pl.BlockSpec((1, din), lambda i: (0, 0)),
            pl.BlockSpec((_MB, 1), lambda i: (i, 0)),
            pl.BlockSpec((din, dout), lambda i: (0, 0)),
        ],
        out_specs=pl.BlockSpec((_MB, dout), lambda i: (i, 0)),
        out_shape=jax.ShapeDtypeStruct((NPAD, dout), jnp.float32),
    )(p, nin, b, nout, w)


_FMB = 400


def _final_body(p_ref, nin_ref, b_ref, y_ref):
    agg = p_ref[0] + p_ref[1]
    y_ref[...] = agg * nin_ref[...] + b_ref[...]


def _final(p, nin, b):
    d = p.shape[-1]
    return pl.pallas_call(
        _final_body,
        grid=(N // _FMB,),
        in_specs=[
            pl.BlockSpec((NC, _FMB, d), lambda i: (0, i, 0)),
            pl.BlockSpec((_FMB, 1), lambda i: (i, 0)),
            pl.BlockSpec((1, d), lambda i: (0, 0)),
        ],
        out_specs=pl.BlockSpec((_FMB, d), lambda i: (i, 0)),
        out_shape=jax.ShapeDtypeStruct((N, d), jnp.float32),
    )(p, nin, b)


def kernel(inputs, edge_index, W0, b0, W1, b1, W2, b2):
    ei = edge_index.astype(jnp.int32)
    pad = jnp.full((2, EPAD - E), N, jnp.int32)
    ei = jnp.concatenate([ei, pad], axis=1)
    src = ei[0].reshape(ECH, CH)
    dst = ei[1].reshape(ECH, CH)
    src64 = ei[0].reshape(ECH * 2, CH // 2)
    dst64 = ei[1].reshape(ECH * 2, CH // 2)

    xp = jnp.pad(inputs, ((0, NPAD - N), (0, 0)))

    dp = _make_deg()(src, dst)
    nout, nin = _norms(dp)

    agg128 = _make_agg(128, 64, AGG128_N0, AGG128_N1)
    y0 = _mm_first(xp, nout, W0)
    p0 = agg128(y0, src64, dst64)
    y1 = _mm_mid(p0, nin, b0.reshape(1, -1), nout, W1)
    p1 = agg128(y1, src64, dst64)
    y2 = _mm_mid(p1, nin, b1.reshape(1, -1), nout, W2)
    p2 = _make_agg(64, 128, AGG64_N0, AGG64_N1)(y2, src, dst)
    return _final(p2, nin, b2.reshape(1, -1))


# speedup vs baseline: 1.2630x; 1.0217x over previous
import functools

import jax
import jax.numpy as jnp
from jax import lax
from jax.experimental import pallas as pl
from jax.experimental.pallas import tpu as pltpu
from jax.experimental.pallas import tpu_sc as plsc

N = 10000
E = 320000
NC = 2
NS = 16
NW = NC * NS
CH = 128
NCH = 79
ECH = NW * NCH
EPAD = ECH * CH
NPAD = 10240
ROWS_PER_TILE = NPAD // NS
DW = 16

_SC_PARAMS = pltpu.CompilerParams(use_tc_tiling_on_sc=False)

AGG128_N0, AGG128_N1 = 222, 94
AGG64_N0, AGG64_N1 = 111, 47


@functools.lru_cache(maxsize=None)
def _mesh():
    return plsc.VectorSubcoreMesh(
        core_axis_name="c", subcore_axis_name="s",
        num_cores=NC, num_subcores=NS)


@functools.lru_cache(maxsize=None)
def _make_deg():
    return functools.partial(
        pl.kernel,
        out_type=jax.ShapeDtypeStruct((NC, NPAD, DW), jnp.float32),
        mesh=_mesh(),
        scratch_types=[
            pltpu.VMEM((NCH, CH), jnp.int32),
            pltpu.VMEM((NCH, CH), jnp.int32),
            pltpu.VMEM((CH, DW), jnp.float32),
            pltpu.VMEM((CH, DW), jnp.float32),
            pltpu.VMEM((ROWS_PER_TILE, DW), jnp.float32),
            pltpu.VMEM_SHARED((NPAD, DW), jnp.float32),
            pltpu.SemaphoreType.DMA,
            pltpu.SemaphoreType.DMA,
        ],
        compiler_params=_SC_PARAMS,
    )(_deg_body)


def _deg_body(src_hbm, dst_hbm, out_hbm, srcall, dstall, oneh0, oneh1, zbuf,
              dacc, semA, semB):
    cid = lax.axis_index("c")
    sid = lax.axis_index("s")
    wid = sid * NC + cid

    lane = lax.iota(jnp.int32, DW)
    e0 = jnp.where(lane == 0, 1.0, 0.0).astype(jnp.float32)
    e1 = jnp.where(lane == 1, 1.0, 0.0).astype(jnp.float32)
    z = jnp.zeros((DW,), jnp.float32)

    @pl.loop(0, CH)
    def _(r):
        oneh0[r, :] = e0
        oneh1[r, :] = e1

    @pl.loop(0, ROWS_PER_TILE)
    def _(r):
        zbuf[r, :] = z

    r0 = sid * ROWS_PER_TILE
    pltpu.sync_copy(zbuf, dacc.at[pl.ds(r0, ROWS_PER_TILE)])

    cbase = wid * NCH
    pltpu.sync_copy(src_hbm.at[pl.ds(cbase, NCH), :], srcall)
    pltpu.sync_copy(dst_hbm.at[pl.ds(cbase, NCH), :], dstall)
    plsc.subcore_barrier()

    def startA(c):
        pltpu.make_async_copy(oneh0, dacc.at[srcall.at[c]], semA).start(
            add=True)

    def startB(c):
        pltpu.make_async_copy(oneh1, dacc.at[dstall.at[c]], semB).start(
            add=True)

    def waitA(c):
        pltpu.make_async_copy(oneh0, dacc.at[srcall.at[c]], semA).wait()

    def waitB(c):
        pltpu.make_async_copy(oneh1, dacc.at[dstall.at[c]], semB).wait()

    startA(0)
    startB(0)

    @pl.loop(0, NCH - 1)
    def _(c):
        startA(c + 1)
        startB(c + 1)
        waitA(c)
        waitB(c)

    waitA(NCH - 1)
    waitB(NCH - 1)

    plsc.subcore_barrier()

    @pl.when(cid == 0)
    def _():
        pltpu.sync_copy(dacc.at[pl.ds(r0, ROWS_PER_TILE)],
                        out_hbm.at[0, pl.ds(r0, ROWS_PER_TILE), :])

    @pl.when(cid == 1)
    def _():
        pltpu.sync_copy(dacc.at[pl.ds(r0, ROWS_PER_TILE)],
                        out_hbm.at[1, pl.ds(r0, ROWS_PER_TILE), :])


@functools.lru_cache(maxsize=None)
def _make_agg(d, ch, n0, n1):
    assert NS * (n0 + n1) == EPAD // ch
    nmax = max(n0, n1)

    @functools.partial(
        pl.kernel,
        out_type=jax.ShapeDtypeStruct((NC, NPAD, d), jnp.float32),
        mesh=_mesh(),
        scratch_types=[
            pltpu.VMEM((nmax, ch), jnp.int32),
            pltpu.VMEM((ch,), jnp.int32),
            pltpu.VMEM((ch,), jnp.int32),
            pltpu.VMEM((ch, d), jnp.float32),
            pltpu.VMEM((ch, d), jnp.float32),
            pltpu.VMEM_SHARED((NPAD, d), jnp.float32),
            pltpu.SemaphoreType.DMA,
            pltpu.SemaphoreType.DMA,
            pltpu.SemaphoreType.DMA,
            pltpu.SemaphoreType.DMA,
        ],
        compiler_params=_SC_PARAMS,
    )
    def agg(y_hbm, src_hbm, dst_hbm, out_hbm, srcall, dstb0, dstb1, rows0,
            rows1, acc, gsem0, gsem1, dsem0, dsem1):
        cid = lax.axis_index("c")
        sid = lax.axis_index("s")

        z = jnp.zeros((16,), jnp.float32)

        @pl.loop(0, ch)
        def _(r):
            @pl.loop(0, d // 16)
            def _(j):
                rows0[r, pl.ds(j * 16, 16)] = z

        r0 = sid * ROWS_PER_TILE
        @pl.loop(0, ROWS_PER_TILE // ch)
        def _(k):
            pltpu.sync_copy(rows0, acc.at[pl.ds(r0 + k * ch, ch)])

        def run(cbase, count):
            pltpu.sync_copy(src_hbm.at[pl.ds(cbase, count), :],
                            srcall.at[pl.ds(0, count), :])

            def gstart(c, rows, sem):
                pltpu.make_async_copy(
                    y_hbm.at[srcall.at[c]], rows, sem).start()

            def gwait(c, rows, sem):
                pltpu.make_async_copy(y_hbm.at[srcall.at[c]], rows, sem).wait()

            def dstart(c, dstb, sem):
                pltpu.make_async_copy(
                    dst_hbm.at[cbase + c], dstb, sem).start()

            def dwait(c, dstb, sem):
                pltpu.make_async_copy(dst_hbm.at[cbase + c], dstb, sem).wait()

            def scat(rows, dstb):
                pltpu.sync_copy(rows, acc.at[dstb], add=True)

            dstart(0, dstb0, dsem0)
            gstart(0, rows0, gsem0)
            if count > 1:
                dstart(1, dstb1, dsem1)
                gstart(1, rows1, gsem1)

            @pl.loop(0, count // 2)
            def _(p):
                c0 = 2 * p
                gwait(c0, rows0, gsem0)
                dwait(c0, dstb0, dsem0)
                scat(rows0, dstb0)

                @pl.when(c0 + 2 < count)
                def _():
                    dstart(c0 + 2, dstb0, dsem0)
                    gstart(c0 + 2, rows0, gsem0)

                gwait(c0 + 1, rows1, gsem1)
                dwait(c0 + 1, dstb1, dsem1)
                scat(rows1, dstb1)

                @pl.when(c0 + 3 < count)
                def _():
                    dstart(c0 + 3, dstb1, dsem1)
                    gstart(c0 + 3, rows1, gsem1)

            if count % 2:
                gwait(count - 1, rows0, gsem0)
                dwait(count - 1, dstb0, dsem0)
                scat(rows0, dstb0)

        plsc.subcore_barrier()

        @pl.when(cid == 0)
        def _():
            run(sid * n0, n0)

        @pl.when(cid == 1)
        def _():
            run(NS * n0 + sid * n1, n1)

        plsc.subcore_barrier()

        @pl.when(cid == 0)
        def _():
            pltpu.sync_copy(acc.at[pl.ds(r0, ROWS_PER_TILE)],
                            out_hbm.at[0, pl.ds(r0, ROWS_PER_TILE), :])

        @pl.when(cid == 1)
        def _():
            pltpu.sync_copy(acc.at[pl.ds(r0, ROWS_PER_TILE)],
                            out_hbm.at[1, pl.ds(r0, ROWS_PER_TILE), :])

    return agg


_MB = 512
_GRIDP = NPAD // _MB


def _mm_first_body(x_ref, dp_ref, w_ref, y_ref, nout_ref, nin_ref):
    s = dp_ref[0] + dp_ref[1]
    dout = s[:, 0:1]
    din = s[:, 1:2]
    nout = jnp.where(dout > 0, lax.rsqrt(dout), 0.0)
    nout_ref[...] = nout
    nin_ref[...] = jnp.where(din > 0, lax.rsqrt(din), 0.0)
    h = x_ref[...] * nout
    y_ref[...] = jnp.dot(h, w_ref[...], preferred_element_type=jnp.float32)


def _mm_first(x, dp, w):
    din, dout = w.shape
    return pl.pallas_call(
        _mm_first_body,
        grid=(_GRIDP,),
        in_specs=[
            pl.BlockSpec((_MB, din), lambda i: (i, 0)),
            pl.BlockSpec((NC, _MB, DW), lambda i: (0, i, 0)),
            pl.BlockSpec((din, dout), lambda i: (0, 0)),
        ],
        out_specs=[
            pl.BlockSpec((_MB, dout), lambda i: (i, 0)),
            pl.BlockSpec((_MB, 1), lambda i: (i, 0)),
            pl.BlockSpec((_MB, 1), lambda i: (i, 0)),
        ],
        out_shape=[
            jax.ShapeDtypeStruct((NPAD, dout), jnp.float32),
            jax.ShapeDtypeStruct((NPAD, 1), jnp.float32),
            jax.ShapeDtypeStruct((NPAD, 1), jnp.float32),
        ],
    )(x, dp, w)


def _mm_mid_body(p_ref, nin_ref, b_ref, nout_ref, w_ref, y_ref):
    agg = p_ref[0] + p_ref[1]
    h = jax.nn.relu(agg * nin_ref[...] + b_ref[...])
    h = h * nout_ref[...]
    y_ref[...] = jnp.dot(h, w_ref[...], preferred_element_type=jnp.float32)


def _mm_mid(p, nin, b, nout, w):
    din, dout = w.shape
    return pl.pallas_call(
        _mm_mid_body,
        grid=(_GRIDP,),
        in_specs=[
            pl.BlockSpec((NC, _MB, din), lambda i: (0, i, 0)),
            pl.BlockSpec((_MB, 1), lambda i: (i, 0)),
            pl.BlockSpec((1, din), lambda i: (0, 0)),
            pl.BlockSpec((_MB, 1), lambda i: (i, 0)),
            pl.BlockSpec((din, dout), lambda i: (0, 0)),
        ],
        out_specs=pl.BlockSpec((_MB, dout), lambda i: (i, 0)),
        out_shape=jax.ShapeDtypeStruct((NPAD, dout), jnp.float32),
    )(p, nin, b, nout, w)


_FMB = 400


def _final_body(p_ref, nin_ref, b_ref, y_ref):
    agg = p_ref[0] + p_ref[1]
    y_ref[...] = agg * nin_ref[...] + b_ref[...]


def _final(p, nin, b):
    d = p.shape[-1]
    return pl.pallas_call(
        _final_body,
        grid=(N // _FMB,),
        in_specs=[
            pl.BlockSpec((NC, _FMB, d), lambda i: (0, i, 0)),
            pl.BlockSpec((_FMB, 1), lambda i: (i, 0)),
            pl.BlockSpec((1, d), lambda i: (0, 0)),
        ],
        out_specs=pl.BlockSpec((_FMB, d), lambda i: (i, 0)),
        out_shape=jax.ShapeDtypeStruct((N, d), jnp.float32),
    )(p, nin, b)


def kernel(inputs, edge_index, W0, b0, W1, b1, W2, b2):
    ei = edge_index.astype(jnp.int32)
    pad = jnp.full((2, EPAD - E), N, jnp.int32)
    ei = jnp.concatenate([ei, pad], axis=1)
    src = ei[0].reshape(ECH, CH)
    dst = ei[1].reshape(ECH, CH)
    src64 = ei[0].reshape(ECH * 2, CH // 2)
    dst64 = ei[1].reshape(ECH * 2, CH // 2)

    xp = jnp.pad(inputs, ((0, NPAD - N), (0, 0)))

    dp = _make_deg()(src, dst)

    agg128 = _make_agg(128, 64, AGG128_N0, AGG128_N1)
    y0, nout, nin = _mm_first(xp, dp, W0)
    p0 = agg128(y0, src64, dst64)
    y1 = _mm_mid(p0, nin, b0.reshape(1, -1), nout, W1)
    p1 = agg128(y1, src64, dst64)
    y2 = _mm_mid(p1, nin, b1.reshape(1, -1), nout, W2)
    p2 = _make_agg(64, 128, AGG64_N0, AGG64_N1)(y2, src, dst)
    return _final(p2, nin, b2.reshape(1, -1))
